# Initial kernel scaffold; baseline (speedup 1.0000x reference)
import jax
import jax.numpy as jnp
from jax.experimental import pallas as pl


def kernel(x, edge_index, batch, W1, b1, W2, b2, Wf, bf):
    raise NotImplementedError("write your pallas kernel here")


# speedup vs baseline: 19.0880x; 19.0880x over previous
import jax
import jax.numpy as jnp
from jax import lax
from jax.experimental import pallas as pl
from jax.experimental.pallas import tpu as pltpu
from jax.experimental.pallas import tpu_sc as plsc

N = 10000
E = 320000
G = 256

NC = 2
NS = 16
NW = NC * NS
EPW = E // NW
CH = 100
NJ = EPW // CH
NP = 10112
RPS = NP // NS
DEGW = 128


def _sc_mesh():
    return plsc.VectorSubcoreMesh(core_axis_name="c", subcore_axis_name="s")


def _deg_body(dst_hbm, ones_hbm, zeros_hbm, out_hbm, idx_v, val_v, acc_sh, sem):
    c = lax.axis_index("c")
    s = lax.axis_index("s")
    wid = s * NC + c
    pltpu.sync_copy(zeros_hbm.at[pl.ds(s * RPS, RPS)], acc_sh.at[pl.ds(s * RPS, RPS)])
    pltpu.sync_copy(ones_hbm, val_v)
    pltpu.sync_copy(dst_hbm.at[wid], idx_v)
    plsc.subcore_barrier()

    def body(j, carry):
        pltpu.sync_copy(val_v, acc_sh.at[idx_v.at[j]], add=True)
        return carry

    lax.fori_loop(0, NJ, body, 0)
    plsc.subcore_barrier()
    pltpu.sync_copy(acc_sh.at[pl.ds(s * RPS, RPS)],
                    out_hbm.at[c].at[pl.ds(s * RPS, RPS)])


_deg_call = pl.kernel(
    _deg_body,
    mesh=_sc_mesh(),
    out_type=jax.ShapeDtypeStruct((NC, NP, DEGW), jnp.float32),
    scratch_types=[
        pltpu.VMEM((NJ, CH), jnp.int32),
        pltpu.VMEM((CH, DEGW), jnp.float32),
        pltpu.VMEM_SHARED((NP, DEGW), jnp.float32),
        pltpu.SemaphoreType.DMA,
    ],
)


def _scatter_body(y_hbm, src_hbm, dst_hbm, zeros_hbm, out_hbm,
                  sidx_v, didx_v, rows_v, acc_sh, sem):
    c = lax.axis_index("c")
    s = lax.axis_index("s")
    wid = s * NC + c
    pltpu.sync_copy(zeros_hbm.at[pl.ds(s * RPS, RPS)], acc_sh.at[pl.ds(s * RPS, RPS)])
    pltpu.sync_copy(src_hbm.at[wid], sidx_v)
    pltpu.sync_copy(dst_hbm.at[wid], didx_v)
    plsc.subcore_barrier()

    def body(j, carry):
        pltpu.async_copy(y_hbm.at[sidx_v.at[j]], rows_v, sem).wait()
        pltpu.sync_copy(rows_v, acc_sh.at[didx_v.at[j]], add=True)
        return carry

    lax.fori_loop(0, NJ, body, 0)
    plsc.subcore_barrier()
    pltpu.sync_copy(acc_sh.at[pl.ds(s * RPS, RPS)],
                    out_hbm.at[c].at[pl.ds(s * RPS, RPS)])


def _make_scatter(D):
    return pl.kernel(
        _scatter_body,
        mesh=_sc_mesh(),
        out_type=jax.ShapeDtypeStruct((NC, NP, D), jnp.float32),
        scratch_types=[
            pltpu.VMEM((NJ, CH), jnp.int32),
            pltpu.VMEM((NJ, CH), jnp.int32),
            pltpu.VMEM((CH, D), jnp.float32),
            pltpu.VMEM_SHARED((NP, D), jnp.float32),
            pltpu.SemaphoreType.DMA,
        ],
    )


_scatter128 = _make_scatter(128)


def _dinv(dega_ref, degb_ref):
    deg = dega_ref[:, 0:1] + degb_ref[:, 0:1] + 1.0
    return lax.rsqrt(deg)


def _y1_body(x_ref, w1_ref, dega_ref, degb_ref, y1_ref):
    dinv = _dinv(dega_ref, degb_ref)
    xw = jnp.dot(x_ref[...], w1_ref[...], preferred_element_type=jnp.float32)
    y1_ref[...] = xw * dinv


def _h1y2_body(a0_ref, a1_ref, y1_ref, dega_ref, degb_ref, w2_ref, b1_ref,
               y2_ref):
    dinv = _dinv(dega_ref, degb_ref)
    t = (a0_ref[...] + a1_ref[...] + y1_ref[...]) * dinv
    h1 = jnp.maximum(t[:, :64] + b1_ref[...], 0.0)
    y2_ref[...] = jnp.dot(h1, w2_ref[...],
                          preferred_element_type=jnp.float32) * dinv


def _final_body(a0_ref, a1_ref, y2_ref, dega_ref, degb_ref, b2_ref,
                batch_ref, wf_ref, bf_ref, out_ref):
    dinv = _dinv(dega_ref, degb_ref)
    h2 = jnp.maximum((a0_ref[...] + a1_ref[...] + y2_ref[...]) * dinv
                     + b2_ref[...], 0.0)
    gid = lax.broadcasted_iota(jnp.int32, (G, N), 0)
    sel = (batch_ref[...] == gid).astype(jnp.float32)
    sums = jnp.dot(sel, h2, preferred_element_type=jnp.float32)
    counts = jnp.sum(sel, axis=1, keepdims=True)
    pooled = sums / jnp.maximum(counts, 1.0)
    out_ref[...] = jnp.dot(pooled, wf_ref[...],
                           preferred_element_type=jnp.float32) + bf_ref[...]


def kernel(x, edge_index, batch, W1, b1, W2, b2, Wf, bf):
    src = edge_index[0].reshape(NW, NJ, CH)
    dst = edge_index[1].reshape(NW, NJ, CH)
    ones_rows = jnp.ones((CH, DEGW), jnp.float32)
    z128 = jnp.zeros((NP, 128), jnp.float32)
    w1p = jnp.pad(W1, ((0, 0), (0, 64)))

    degp = _deg_call(dst, ones_rows, z128)
    dega, degb = degp[0, :N], degp[1, :N]

    y1 = pl.pallas_call(
        _y1_body,
        out_shape=jax.ShapeDtypeStruct((N, 128), jnp.float32),
    )(x, w1p, dega, degb)

    agg1 = _scatter128(y1, src, dst, z128)
    a1a, a1b = agg1[0, :N], agg1[1, :N]

    y2 = pl.pallas_call(
        _h1y2_body,
        out_shape=jax.ShapeDtypeStruct((N, 128), jnp.float32),
    )(a1a, a1b, y1, dega, degb, W2, b1.reshape(1, 64))

    agg2 = _scatter128(y2, src, dst, z128)
    a2a, a2b = agg2[0, :N], agg2[1, :N]

    out = pl.pallas_call(
        _final_body,
        out_shape=jax.ShapeDtypeStruct((G, 128), jnp.float32),
    )(a2a, a2b, y2, dega, degb, b2.reshape(1, 128),
      batch.reshape(1, N), Wf, bf.reshape(1, 128))
    return out
